# Initial kernel scaffold; baseline (speedup 1.0000x reference)
import jax
import jax.numpy as jnp
from jax.experimental import pallas as pl


def kernel(h, e, edge_index, proj_W, proj_b, edge_W1, edge_b1, edge_W2, edge_b2, gnn_b, W_ih, W_hh, b_ih, b_hh):
    raise NotImplementedError("write your pallas kernel here")


# speedup vs baseline: 1.5813x; 1.5813x over previous
import functools

import jax
import jax.numpy as jnp
from jax import lax
from jax.experimental import pallas as pl
from jax.experimental.pallas import tpu as pltpu
from jax.experimental.pallas import tpu_sc as plsc

_NC, _NS = 2, 16
_NW = _NC * _NS
_CH = 128
_STEPS = 3


def _proj_body(h_ref, wt_ref, b_ref, out_ref):
    acc = jnp.dot(h_ref[...], wt_ref[...], preferred_element_type=jnp.float32)
    v = jnp.maximum(acc + b_ref[...], 0.0)
    out_ref[...] = jnp.concatenate([v, jnp.zeros(v.shape, v.dtype)], axis=1)


def _ewt_body(et_ref, w1_ref, b1_ref, w2_ref, b2_ref, out_ref):
    ehidt = jnp.dot(w1_ref[...], et_ref[...], preferred_element_type=jnp.float32)
    ehidt = jnp.maximum(ehidt + b1_ref[...], 0.0)
    ewt = jnp.dot(w2_ref[...], ehidt.astype(jnp.bfloat16),
                  preferred_element_type=jnp.float32) + b2_ref[...]
    out_ref[...] = ewt.astype(jnp.bfloat16)


def _msg_body(n_real, d_out, hs_ref, ewt_ref, out_ref):
    pid = pl.program_id(0)

    @pl.when(pid < n_real)
    def _():
        hst = jnp.swapaxes(hs_ref[...], 0, 1)
        be = hst.shape[1]
        ew4 = ewt_ref[...].astype(jnp.float32).reshape(d_out, d_out, be)
        mt = jnp.sum(ew4 * hst[0:d_out].reshape(d_out, 1, be), axis=0)
        m64 = jnp.swapaxes(mt, 0, 1)
        out_ref[...] = jnp.concatenate(
            [m64, jnp.zeros(m64.shape, m64.dtype)], axis=1)

    @pl.when(pid >= n_real)
    def _():
        out_ref[...] = jnp.zeros(out_ref.shape, out_ref.dtype)


def _gru_body(d_out, part_ref, ht_ref, gb_ref, wiht_ref, whht_ref,
              bih_ref, bhh_ref, out_ref):
    a = jnp.maximum(part_ref[0][:, 0:d_out] + part_ref[1][:, 0:d_out]
                    + gb_ref[...], 0.0)
    ht = ht_ref[:, 0:d_out]
    gi = jnp.dot(a, wiht_ref[...], preferred_element_type=jnp.float32) + bih_ref[...]
    gh = jnp.dot(ht, whht_ref[...], preferred_element_type=jnp.float32) + bhh_ref[...]
    r = jax.nn.sigmoid(gi[:, 0:d_out] + gh[:, 0:d_out])
    z = jax.nn.sigmoid(gi[:, d_out:2 * d_out] + gh[:, d_out:2 * d_out])
    n = jnp.tanh(gi[:, 2 * d_out:3 * d_out] + r * gh[:, 2 * d_out:3 * d_out])
    hnew = (1.0 - z) * n + z * ht
    out_ref[...] = jnp.concatenate([hnew, jnp.zeros(hnew.shape, hnew.dtype)],
                                   axis=1)


def _make_gather(n_nodes, d, e_pad, per_tile):
    n_chunks = per_tile // _CH
    mesh = plsc.VectorSubcoreMesh(core_axis_name="c", subcore_axis_name="s")

    @functools.partial(
        pl.kernel, mesh=mesh,
        out_type=jax.ShapeDtypeStruct((e_pad, d), jnp.float32),
        scratch_types=[
            pltpu.VMEM((n_chunks, _CH), jnp.int32),
            pltpu.VMEM((_CH, d), jnp.float32),
            pltpu.SemaphoreType.DMA,
        ],
    )
    def gather(table_hbm, src2d_hbm, out_hbm, idx_v, rows_v, sem):
        wid = lax.axis_index("s") * _NC + lax.axis_index("c")
        pltpu.sync_copy(src2d_hbm.at[pl.ds(wid * n_chunks, n_chunks)], idx_v)

        def body(j, carry):
            pltpu.async_copy(table_hbm.at[idx_v.at[j]], rows_v, sem).wait()
            pltpu.sync_copy(
                rows_v, out_hbm.at[pl.ds(wid * per_tile + j * _CH, _CH)])
            return carry

        lax.fori_loop(0, n_chunks, body, 0)

    return gather


def _make_scatter(n_nodes, d, e_pad, per_tile):
    n_chunks = per_tile // _CH
    rows_per_tile = n_nodes // _NS
    mesh = plsc.VectorSubcoreMesh(core_axis_name="c", subcore_axis_name="s")

    @functools.partial(
        pl.kernel, mesh=mesh,
        out_type=jax.ShapeDtypeStruct((_NC * n_nodes, d), jnp.float32),
        scratch_types=[
            pltpu.VMEM((n_chunks, _CH), jnp.int32),
            pltpu.VMEM((_CH, d), jnp.float32),
            pltpu.VMEM_SHARED((n_nodes, d), jnp.float32),
        ],
    )
    def scatter(m_hbm, dst2d_hbm, zeros_hbm, out_hbm, idx_v, rows_v, acc):
        cid = lax.axis_index("c")
        sid = lax.axis_index("s")
        wid = sid * _NC + cid
        pltpu.sync_copy(zeros_hbm, acc.at[pl.ds(sid * rows_per_tile, rows_per_tile)])
        plsc.subcore_barrier()
        pltpu.sync_copy(dst2d_hbm.at[pl.ds(wid * n_chunks, n_chunks)], idx_v)

        def body(j, carry):
            pltpu.sync_copy(
                m_hbm.at[pl.ds(wid * per_tile + j * _CH, _CH)], rows_v)
            pltpu.sync_copy(rows_v, acc.at[idx_v.at[j]], add=True)
            return carry

        lax.fori_loop(0, n_chunks, body, 0)
        plsc.subcore_barrier()
        pltpu.sync_copy(
            acc.at[pl.ds(sid * rows_per_tile, rows_per_tile)],
            out_hbm.at[pl.ds(cid * n_nodes + sid * rows_per_tile, rows_per_tile)])

    return scatter


def kernel(h, e, edge_index, proj_W, proj_b, edge_W1, edge_b1, edge_W2,
           edge_b2, gnn_b, W_ih, W_hh, b_ih, b_hh):
    n_nodes, d_in = h.shape
    n_edges, d_e = e.shape
    d_out = proj_W.shape[0]
    d_eh = edge_W1.shape[0]

    per_tile = -(-n_edges // (_NW * _CH)) * _CH
    e_pad = per_tile * _NW
    pad = e_pad - n_edges

    n_pad = -(-n_nodes // (_NS * 8)) * (_NS * 8)

    src2d = jnp.concatenate(
        [edge_index[0], jnp.zeros((pad,), jnp.int32)]).reshape(-1, _CH)
    dst2d = jnp.concatenate(
        [edge_index[1], jnp.zeros((pad,), jnp.int32)]).reshape(-1, _CH)
    e_padded = jnp.concatenate([e, jnp.zeros((pad, d_e), e.dtype)])
    zeros_stripe = jnp.zeros((n_pad // _NS, 2 * d_out), jnp.float32)

    bn = 2000 if n_nodes % 2000 == 0 else n_nodes
    hcur = pl.pallas_call(
        _proj_body,
        grid=(n_nodes // bn,),
        in_specs=[
            pl.BlockSpec((bn, d_in), lambda i: (i, 0)),
            pl.BlockSpec((d_in, d_out), lambda i: (0, 0)),
            pl.BlockSpec((1, d_out), lambda i: (0, 0)),
        ],
        out_specs=pl.BlockSpec((bn, 2 * d_out), lambda i: (i, 0)),
        out_shape=jax.ShapeDtypeStruct((n_nodes, 2 * d_out), jnp.float32),
    )(h, proj_W.T, proj_b.reshape(1, d_out))

    be = next(b for b in (640, 512, 256, 128)
              if e_pad % b == 0 and n_edges % b == 0)
    n_blocks = e_pad // be
    n_real = n_edges // be
    ewt = pl.pallas_call(
        _ewt_body,
        grid=(n_blocks,),
        in_specs=[
            pl.BlockSpec((d_e, be), lambda i: (0, i)),
            pl.BlockSpec((d_eh, d_e), lambda i: (0, 0)),
            pl.BlockSpec((d_eh, 1), lambda i: (0, 0)),
            pl.BlockSpec((d_out * d_out, d_eh), lambda i: (0, 0)),
            pl.BlockSpec((d_out * d_out, 1), lambda i: (0, 0)),
        ],
        out_specs=pl.BlockSpec((d_out * d_out, be), lambda i: (0, i)),
        out_shape=jax.ShapeDtypeStruct((d_out * d_out, e_pad), jnp.bfloat16),
    )(e_padded.T, edge_W1, edge_b1.reshape(d_eh, 1),
      edge_W2.astype(jnp.bfloat16), edge_b2.reshape(d_out * d_out, 1))

    gather = _make_gather(n_nodes, 2 * d_out, e_pad, per_tile)
    scatter = _make_scatter(n_pad, 2 * d_out, e_pad, per_tile)

    msg = pl.pallas_call(
        functools.partial(_msg_body, n_real, d_out),
        grid=(n_blocks,),
        in_specs=[
            pl.BlockSpec((be, 2 * d_out), lambda i: (i, 0)),
            pl.BlockSpec((d_out * d_out, be), lambda i: (0, i)),
        ],
        out_specs=pl.BlockSpec((be, 2 * d_out), lambda i: (i, 0)),
        out_shape=jax.ShapeDtypeStruct((e_pad, 2 * d_out), jnp.float32),
    )

    gru = pl.pallas_call(
        functools.partial(_gru_body, d_out),
        grid=(n_nodes // bn,),
        in_specs=[
            pl.BlockSpec((_NC, bn, 2 * d_out), lambda i: (0, i, 0)),
            pl.BlockSpec((bn, 2 * d_out), lambda i: (i, 0)),
            pl.BlockSpec((1, d_out), lambda i: (0, 0)),
            pl.BlockSpec((d_out, 3 * d_out), lambda i: (0, 0)),
            pl.BlockSpec((d_out, 3 * d_out), lambda i: (0, 0)),
            pl.BlockSpec((1, 3 * d_out), lambda i: (0, 0)),
            pl.BlockSpec((1, 3 * d_out), lambda i: (0, 0)),
        ],
        out_specs=pl.BlockSpec((bn, 2 * d_out), lambda i: (i, 0)),
        out_shape=jax.ShapeDtypeStruct((n_nodes, 2 * d_out), jnp.float32),
    )

    wiht = W_ih.T
    whht = W_hh.T
    bih2 = b_ih.reshape(1, 3 * d_out)
    bhh2 = b_hh.reshape(1, 3 * d_out)
    gb2 = gnn_b.reshape(1, d_out)

    ht = hcur
    for _ in range(_STEPS):
        hs = gather(hcur, src2d)
        m = msg(hs, ewt)
        part = scatter(m, dst2d, zeros_stripe)
        hcur = gru(part.reshape(_NC, n_pad, 2 * d_out), ht, gb2, wiht, whht,
                   bih2, bhh2)
        ht = hcur
    return hcur[:, 0:d_out]


# speedup vs baseline: 1.6662x; 1.0537x over previous
import functools

import jax
import jax.numpy as jnp
from jax import lax
from jax.experimental import pallas as pl
from jax.experimental.pallas import tpu as pltpu
from jax.experimental.pallas import tpu_sc as plsc

_NC, _NS = 2, 16
_NW = _NC * _NS
_CH = 128
_STEPS = 3


def _proj_body(h_ref, wt_ref, b_ref, out_ref):
    acc = jnp.dot(h_ref[...], wt_ref[...], preferred_element_type=jnp.float32)
    v = jnp.maximum(acc + b_ref[...], 0.0)
    out_ref[...] = jnp.concatenate([v, jnp.zeros(v.shape, v.dtype)], axis=1)


def _ewt_body(et_ref, w1_ref, b1_ref, w2_ref, b2_ref, out_ref):
    ehidt = jnp.dot(w1_ref[...], et_ref[...], preferred_element_type=jnp.float32)
    ehidt = jnp.maximum(ehidt + b1_ref[...], 0.0)
    ewt = jnp.dot(w2_ref[...], ehidt.astype(jnp.bfloat16),
                  preferred_element_type=jnp.float32) + b2_ref[...]
    out_ref[...] = ewt.astype(jnp.bfloat16)


def _msg_body(n_real, d_out, hs_ref, ewt_ref, out_ref):
    pid = pl.program_id(0)

    @pl.when(pid < n_real)
    def _():
        hst = jnp.swapaxes(hs_ref[...], 0, 1)
        be = hst.shape[1]
        ew4 = ewt_ref[...].astype(jnp.float32).reshape(d_out, d_out, be)
        mt = jnp.sum(ew4 * hst[0:d_out].reshape(d_out, 1, be), axis=0)
        m64 = jnp.swapaxes(mt, 0, 1)
        out_ref[...] = jnp.concatenate(
            [m64, jnp.zeros(m64.shape, m64.dtype)], axis=1)

    @pl.when(pid >= n_real)
    def _():
        out_ref[...] = jnp.zeros(out_ref.shape, out_ref.dtype)


def _gru_body(d_out, part_ref, ht_ref, gb_ref, wiht_ref, whht_ref,
              bih_ref, bhh_ref, out_ref):
    a = jnp.maximum(part_ref[0][:, 0:d_out] + part_ref[1][:, 0:d_out]
                    + gb_ref[...], 0.0)
    ht = ht_ref[:, 0:d_out]
    gi = jnp.dot(a, wiht_ref[...], preferred_element_type=jnp.float32) + bih_ref[...]
    gh = jnp.dot(ht, whht_ref[...], preferred_element_type=jnp.float32) + bhh_ref[...]
    r = jax.nn.sigmoid(gi[:, 0:d_out] + gh[:, 0:d_out])
    z = jax.nn.sigmoid(gi[:, d_out:2 * d_out] + gh[:, d_out:2 * d_out])
    n = jnp.tanh(gi[:, 2 * d_out:3 * d_out] + r * gh[:, 2 * d_out:3 * d_out])
    hnew = (1.0 - z) * n + z * ht
    out_ref[...] = jnp.concatenate([hnew, jnp.zeros(hnew.shape, hnew.dtype)],
                                   axis=1)


_NBUF = 4
_SNBUF = 2


def _make_gather(n_nodes, d, e_pad, per_tile):
    n_chunks = per_tile // _CH
    mesh = plsc.VectorSubcoreMesh(core_axis_name="c", subcore_axis_name="s")

    @functools.partial(
        pl.kernel, mesh=mesh,
        out_type=jax.ShapeDtypeStruct((e_pad, d), jnp.float32),
        scratch_types=[
            pltpu.VMEM((n_chunks, _CH), jnp.int32),
        ] + [pltpu.VMEM((_CH, d), jnp.float32) for _ in range(_NBUF)]
          + [pltpu.SemaphoreType.DMA for _ in range(2 * _NBUF)],
    )
    def gather(table_hbm, src2d_hbm, out_hbm, idx_v, *bufs_sems):
        rows = bufs_sems[:_NBUF]
        gsem = bufs_sems[_NBUF:2 * _NBUF]
        osem = bufs_sems[2 * _NBUF:3 * _NBUF]
        wid = lax.axis_index("s") * _NC + lax.axis_index("c")
        base = wid * per_tile
        pltpu.sync_copy(src2d_hbm.at[pl.ds(wid * n_chunks, n_chunks)], idx_v)

        gd = [None] * _NBUF
        od = [None] * _NBUF
        for j in range(n_chunks):
            b = j % _NBUF
            if j >= _NBUF:
                od[b].wait()
            gd[b] = pltpu.async_copy(table_hbm.at[idx_v.at[j]], rows[b],
                                     gsem[b])
            if j >= _NBUF - 1:
                jj = j - (_NBUF - 1)
                bb = jj % _NBUF
                gd[bb].wait()
                od[bb] = pltpu.async_copy(
                    rows[bb], out_hbm.at[pl.ds(base + jj * _CH, _CH)],
                    osem[bb])
        for jj in range(max(0, n_chunks - _NBUF + 1), n_chunks):
            bb = jj % _NBUF
            gd[bb].wait()
            od[bb] = pltpu.async_copy(
                rows[bb], out_hbm.at[pl.ds(base + jj * _CH, _CH)], osem[bb])
        for b in range(min(_NBUF, n_chunks)):
            od[b].wait()

    return gather


def _make_scatter(n_nodes, d, e_pad, per_tile):
    n_chunks = per_tile // _CH
    rows_per_tile = n_nodes // _NS
    mesh = plsc.VectorSubcoreMesh(core_axis_name="c", subcore_axis_name="s")

    @functools.partial(
        pl.kernel, mesh=mesh,
        out_type=jax.ShapeDtypeStruct((_NC * n_nodes, d), jnp.float32),
        scratch_types=[
            pltpu.VMEM((n_chunks, _CH), jnp.int32),
            pltpu.VMEM_SHARED((n_nodes, d), jnp.float32),
        ] + [pltpu.VMEM((_CH, d), jnp.float32) for _ in range(_SNBUF)]
          + [pltpu.SemaphoreType.DMA for _ in range(2 * _SNBUF)],
    )
    def scatter(m_hbm, dst2d_hbm, zeros_hbm, out_hbm, idx_v, acc, *bufs_sems):
        rows = bufs_sems[:_SNBUF]
        isem = bufs_sems[_SNBUF:2 * _SNBUF]
        ssem = bufs_sems[2 * _SNBUF:3 * _SNBUF]
        cid = lax.axis_index("c")
        sid = lax.axis_index("s")
        wid = sid * _NC + cid
        base = wid * per_tile
        pltpu.sync_copy(zeros_hbm,
                        acc.at[pl.ds(sid * rows_per_tile, rows_per_tile)])
        pltpu.sync_copy(dst2d_hbm.at[pl.ds(wid * n_chunks, n_chunks)], idx_v)
        plsc.subcore_barrier()

        idn = [None] * _SNBUF
        sd = [None] * _SNBUF
        for j in range(n_chunks):
            b = j % _SNBUF
            if j >= _SNBUF:
                sd[b].wait()
            idn[b] = pltpu.async_copy(
                m_hbm.at[pl.ds(base + j * _CH, _CH)], rows[b], isem[b])
            if j >= _SNBUF - 1:
                jj = j - (_SNBUF - 1)
                bb = jj % _SNBUF
                idn[bb].wait()
                sd[bb] = pltpu.async_copy(rows[bb], acc.at[idx_v.at[jj]],
                                          ssem[bb], add=True)
        for jj in range(max(0, n_chunks - _SNBUF + 1), n_chunks):
            bb = jj % _SNBUF
            idn[bb].wait()
            sd[bb] = pltpu.async_copy(rows[bb], acc.at[idx_v.at[jj]],
                                      ssem[bb], add=True)
        for b in range(min(_SNBUF, n_chunks)):
            sd[b].wait()
        plsc.subcore_barrier()
        pltpu.sync_copy(
            acc.at[pl.ds(sid * rows_per_tile, rows_per_tile)],
            out_hbm.at[pl.ds(cid * n_nodes + sid * rows_per_tile,
                             rows_per_tile)])

    return scatter


def kernel(h, e, edge_index, proj_W, proj_b, edge_W1, edge_b1, edge_W2,
           edge_b2, gnn_b, W_ih, W_hh, b_ih, b_hh):
    n_nodes, d_in = h.shape
    n_edges, d_e = e.shape
    d_out = proj_W.shape[0]
    d_eh = edge_W1.shape[0]

    per_tile = -(-n_edges // (_NW * _CH)) * _CH
    e_pad = per_tile * _NW
    pad = e_pad - n_edges

    n_pad = -(-n_nodes // (_NS * 8)) * (_NS * 8)

    src2d = jnp.concatenate(
        [edge_index[0], jnp.zeros((pad,), jnp.int32)]).reshape(-1, _CH)
    dst2d = jnp.concatenate(
        [edge_index[1], jnp.zeros((pad,), jnp.int32)]).reshape(-1, _CH)
    e_padded = jnp.concatenate([e, jnp.zeros((pad, d_e), e.dtype)])
    zeros_stripe = jnp.zeros((n_pad // _NS, 2 * d_out), jnp.float32)

    bn = 2000 if n_nodes % 2000 == 0 else n_nodes
    hcur = pl.pallas_call(
        _proj_body,
        grid=(n_nodes // bn,),
        in_specs=[
            pl.BlockSpec((bn, d_in), lambda i: (i, 0)),
            pl.BlockSpec((d_in, d_out), lambda i: (0, 0)),
            pl.BlockSpec((1, d_out), lambda i: (0, 0)),
        ],
        out_specs=pl.BlockSpec((bn, 2 * d_out), lambda i: (i, 0)),
        out_shape=jax.ShapeDtypeStruct((n_nodes, 2 * d_out), jnp.float32),
    )(h, proj_W.T, proj_b.reshape(1, d_out))

    be = next(b for b in (640, 512, 256, 128)
              if e_pad % b == 0 and n_edges % b == 0)
    n_blocks = e_pad // be
    n_real = n_edges // be
    ewt = pl.pallas_call(
        _ewt_body,
        grid=(n_blocks,),
        in_specs=[
            pl.BlockSpec((d_e, be), lambda i: (0, i)),
            pl.BlockSpec((d_eh, d_e), lambda i: (0, 0)),
            pl.BlockSpec((d_eh, 1), lambda i: (0, 0)),
            pl.BlockSpec((d_out * d_out, d_eh), lambda i: (0, 0)),
            pl.BlockSpec((d_out * d_out, 1), lambda i: (0, 0)),
        ],
        out_specs=pl.BlockSpec((d_out * d_out, be), lambda i: (0, i)),
        out_shape=jax.ShapeDtypeStruct((d_out * d_out, e_pad), jnp.bfloat16),
    )(e_padded.T, edge_W1, edge_b1.reshape(d_eh, 1),
      edge_W2.astype(jnp.bfloat16), edge_b2.reshape(d_out * d_out, 1))

    gather = _make_gather(n_nodes, 2 * d_out, e_pad, per_tile)
    scatter = _make_scatter(n_pad, 2 * d_out, e_pad, per_tile)

    msg = pl.pallas_call(
        functools.partial(_msg_body, n_real, d_out),
        grid=(n_blocks,),
        in_specs=[
            pl.BlockSpec((be, 2 * d_out), lambda i: (i, 0)),
            pl.BlockSpec((d_out * d_out, be), lambda i: (0, i)),
        ],
        out_specs=pl.BlockSpec((be, 2 * d_out), lambda i: (i, 0)),
        out_shape=jax.ShapeDtypeStruct((e_pad, 2 * d_out), jnp.float32),
    )

    gru = pl.pallas_call(
        functools.partial(_gru_body, d_out),
        grid=(n_nodes // bn,),
        in_specs=[
            pl.BlockSpec((_NC, bn, 2 * d_out), lambda i: (0, i, 0)),
            pl.BlockSpec((bn, 2 * d_out), lambda i: (i, 0)),
            pl.BlockSpec((1, d_out), lambda i: (0, 0)),
            pl.BlockSpec((d_out, 3 * d_out), lambda i: (0, 0)),
            pl.BlockSpec((d_out, 3 * d_out), lambda i: (0, 0)),
            pl.BlockSpec((1, 3 * d_out), lambda i: (0, 0)),
            pl.BlockSpec((1, 3 * d_out), lambda i: (0, 0)),
        ],
        out_specs=pl.BlockSpec((bn, 2 * d_out), lambda i: (i, 0)),
        out_shape=jax.ShapeDtypeStruct((n_nodes, 2 * d_out), jnp.float32),
    )

    wiht = W_ih.T
    whht = W_hh.T
    bih2 = b_ih.reshape(1, 3 * d_out)
    bhh2 = b_hh.reshape(1, 3 * d_out)
    gb2 = gnn_b.reshape(1, d_out)

    ht = hcur
    for _ in range(_STEPS):
        hs = gather(hcur, src2d)
        m = msg(hs, ewt)
        part = scatter(m, dst2d, zeros_stripe)
        hcur = gru(part.reshape(_NC, n_pad, 2 * d_out), ht, gb2, wiht, whht,
                   bih2, bhh2)
        ht = hcur
    return hcur[:, 0:d_out]
